# baseline (speedup 1.0000x reference)
import functools

import jax
import jax.numpy as jnp
from jax import lax
from jax.experimental import pallas as pl
from jax.experimental.pallas import tpu as pltpu
from jax.experimental.pallas import tpu_sc as plsc

N = 10000
E = 320000
D = 128
DH = D // 2

_f32 = jnp.float32
_i32 = jnp.int32
_bf16 = jnp.bfloat16

_NC = 2
_NS = 16
_NW = _NC * _NS
_EPW = E // _NW
_C = 128
_NCH = 79
_EPWP = _NCH * _C
_NP = 10112
_RPT = _NP // _NS
_PAD_DST = _NP - 1
_ZR = 158
_QB = 6


def _bn_body(x_ref, gamma_ref, beta_ref, h_ref):
    x = x_ref[...]
    mean = jnp.mean(x, axis=0, keepdims=True)
    xc = x - mean
    var = jnp.mean(xc * xc, axis=0, keepdims=True)
    h_ref[...] = xc * jax.lax.rsqrt(var + 1e-5) * gamma_ref[...] + beta_ref[...]


def _prep_body(deg16_ref, h_ref, w0_ref, w2_ref,
               isd_ref, g1a_ref, g1b_ref, y0_ref):
    deg = deg16_ref[0, :N, 0:1] + deg16_ref[1, :N, 0:1]
    isd = jnp.where(deg > 0, jax.lax.rsqrt(deg), 0.0)
    isd_ref[...] = isd
    h = h_ref[...]
    g1 = (isd * h).astype(jnp.bfloat16)
    g1a_ref[...] = g1[:, :DH]
    g1b_ref[...] = g1[:, DH:]
    y0_ref[...] = jnp.dot(h, w0_ref[...] - w2_ref[...],
                          preferred_element_type=jnp.float32)


def _mid_body(s1a_ref, s1b_ref, isd_ref, y0_ref, w1_ref,
              g2a_ref, g2b_ref, y01_ref):
    isd = isd_ref[...]
    s1 = jnp.concatenate(
        [s1a_ref[0, :N].astype(jnp.float32)
         + s1a_ref[1, :N].astype(jnp.float32),
         s1b_ref[0, :N].astype(jnp.float32)
         + s1b_ref[1, :N].astype(jnp.float32)], axis=1)
    tx1 = -isd * s1
    g2 = (isd * tx1).astype(jnp.bfloat16)
    g2a_ref[...] = g2[:, :DH]
    g2b_ref[...] = g2[:, DH:]
    y01_ref[...] = y0_ref[...] + jnp.dot(tx1, w1_ref[...],
                                         preferred_element_type=jnp.float32)


def _fin_body(s2a_ref, s2b_ref, isd_ref, y01_ref, w2_ref, b_ref, out_ref):
    s2 = jnp.concatenate(
        [s2a_ref[0, :N].astype(jnp.float32)
         + s2a_ref[1, :N].astype(jnp.float32),
         s2b_ref[0, :N].astype(jnp.float32)
         + s2b_ref[1, :N].astype(jnp.float32)], axis=1)
    tx2s = (-2.0 * isd_ref[...]) * s2
    acc = y01_ref[...] + jnp.dot(tx2s, w2_ref[...],
                                 preferred_element_type=jnp.float32)
    out_ref[...] = jnp.maximum(acc + b_ref[...], 0.0)


def _tc_call(body, out_shapes):
    return pl.pallas_call(body, out_shape=out_shapes)


def _unpack_dst(pk):
    return jnp.bitwise_and(pk, 16383)


def _unpack_src(pk):
    return lax.shift_right_logical(pk, 14)


def _deg_body(pk_hbm, out_hbm, pkv, didx_v, ones_v, zbuf_v, acc_sh):
    cid = lax.axis_index("c")
    sid = lax.axis_index("s")
    wid = sid * _NC + cid
    r0 = sid * _RPT

    pltpu.sync_copy(pk_hbm.at[wid], pkv)

    def fill1(i, c):
        ones_v[i] = jnp.full((16,), 1.0, _f32)
        return c

    lax.fori_loop(0, _C, fill1, 0)

    def fill0(i, c):
        zbuf_v[i] = jnp.zeros((16,), _f32)
        return c

    lax.fori_loop(0, _RPT, fill0, 0)

    def unpack(i, c):
        for k in range(_C // 16):
            pk = pkv[i, pl.ds(k * 16, 16)]
            didx_v[i, pl.ds(k * 16, 16)] = _unpack_dst(pk)
        return c

    lax.fori_loop(0, _NCH, unpack, 0)

    pltpu.sync_copy(zbuf_v, acc_sh.at[pl.ds(r0, _RPT)])
    plsc.subcore_barrier()

    def chunk(j, c):
        pltpu.sync_copy(ones_v, acc_sh.at[didx_v.at[j]], add=True)
        return c

    lax.fori_loop(0, _NCH, chunk, 0)
    plsc.subcore_barrier()
    pltpu.sync_copy(acc_sh.at[pl.ds(r0, _RPT)],
                    out_hbm.at[cid, pl.ds(r0, _RPT)])


def _segsum_body(ga_hbm, gb_hbm, pk_hbm, outa_hbm, outb_hbm,
                 pkv, sidx_v, didx_v, rows_v, zbuf_v,
                 ga_sh, gb_sh, acc_sh, *sems):
    gsems = sems[:_QB]
    ssems = sems[_QB:]
    cid = lax.axis_index("c")
    sid = lax.axis_index("s")
    wid = sid * _NC + cid
    r0 = sid * _RPT

    pltpu.sync_copy(pk_hbm.at[wid], pkv)

    @pl.when(sid == 0)
    def _():
        pltpu.sync_copy(ga_hbm, ga_sh)

    @pl.when(sid == 1)
    def _():
        pltpu.sync_copy(gb_hbm, gb_sh)

    def fill0(i, c):
        for k in range(DH // 32):
            zbuf_v[i, pl.ds(k * 32, 32)] = jnp.zeros((32,), _bf16)
        return c

    lax.fori_loop(0, _ZR, fill0, 0)

    def unpack(i, c):
        for k in range(_C // 16):
            pk = pkv[i, pl.ds(k * 16, 16)]
            sidx_v[i, pl.ds(k * 16, 16)] = _unpack_src(pk)
            didx_v[i, pl.ds(k * 16, 16)] = _unpack_dst(pk)
        return c

    lax.fori_loop(0, _NCH, unpack, 0)

    for q in range(_RPT // _ZR):
        pltpu.sync_copy(zbuf_v, acc_sh.at[pl.ds(r0 + q * _ZR, _ZR)])
    plsc.subcore_barrier()

    def run_half(g_sh, out_hbm, last):
        for b in range(_QB):
            pltpu.async_copy(g_sh.at[sidx_v.at[b]], rows_v.at[b], gsems[b])

        def step(j, c):
            b = lax.rem(j, _QB)
            jj = j + _QB - 2
            for bs in range(_QB):

                @pl.when(b == bs)
                def _():
                    pltpu.make_async_copy(g_sh.at[sidx_v.at[j]],
                                          rows_v.at[bs], gsems[bs]).wait()
                    pltpu.async_copy(rows_v.at[bs], acc_sh.at[didx_v.at[j]],
                                     ssems[bs], add=True)

                @pl.when((jj < _NCH) & (lax.rem(jj, _QB) == bs) & (jj >= _QB))
                def _():
                    pltpu.make_async_copy(rows_v.at[bs],
                                          acc_sh.at[didx_v.at[0]],
                                          ssems[bs]).wait()
                    pltpu.async_copy(g_sh.at[sidx_v.at[jj]], rows_v.at[bs],
                                     gsems[bs])

            return c

        lax.fori_loop(0, _NCH, step, 0)
        for b in range(_QB):
            pltpu.make_async_copy(rows_v.at[b], acc_sh.at[didx_v.at[0]],
                                  ssems[b]).wait()

        plsc.subcore_barrier()
        pltpu.sync_copy(acc_sh.at[pl.ds(r0, _RPT)],
                        out_hbm.at[cid, pl.ds(r0, _RPT)])
        if not last:
            for q in range(_RPT // _ZR):
                pltpu.sync_copy(zbuf_v, acc_sh.at[pl.ds(r0 + q * _ZR, _ZR)])
            plsc.subcore_barrier()

    run_half(ga_sh, outa_hbm, False)
    run_half(gb_sh, outb_hbm, True)


def _sc_mesh():
    return plsc.VectorSubcoreMesh(core_axis_name="c", subcore_axis_name="s")


@functools.lru_cache(maxsize=None)
def _deg_kernel():
    return pl.kernel(
        _deg_body,
        out_type=jax.ShapeDtypeStruct((_NC, _NP, 16), _f32),
        mesh=_sc_mesh(),
        compiler_params=pltpu.CompilerParams(use_tc_tiling_on_sc=False),
        scratch_types=[
            pltpu.VMEM((_NCH, _C), _i32),
            pltpu.VMEM((_NCH, _C), _i32),
            pltpu.VMEM((_C, 16), _f32),
            pltpu.VMEM((_RPT, 16), _f32),
            pltpu.VMEM_SHARED((_NP, 16), _f32),
        ])


@functools.lru_cache(maxsize=None)
def _segsum_kernel():
    return pl.kernel(
        _segsum_body,
        out_type=(jax.ShapeDtypeStruct((_NC, _NP, DH), _bf16),
                  jax.ShapeDtypeStruct((_NC, _NP, DH), _bf16)),
        mesh=_sc_mesh(),
        compiler_params=pltpu.CompilerParams(use_tc_tiling_on_sc=False),
        scratch_types=[
            pltpu.VMEM((_NCH, _C), _i32),
            pltpu.VMEM((_NCH, _C), _i32),
            pltpu.VMEM((_NCH, _C), _i32),
            pltpu.VMEM((_QB, _C, DH), _bf16),
            pltpu.VMEM((_ZR, DH), _bf16),
            pltpu.VMEM_SHARED((N, DH), _bf16),
            pltpu.VMEM_SHARED((N, DH), _bf16),
            pltpu.VMEM_SHARED((_NP, DH), _bf16),
        ] + [pltpu.SemaphoreType.DMA] * (2 * _QB))


@jax.jit
def kernel(x, edge_index, W, b, gamma, beta):
    gamma2 = gamma.reshape(1, D)
    beta2 = beta.reshape(1, D)
    b2 = b.reshape(1, D)

    src = edge_index[0].astype(_i32)
    dst = edge_index[1].astype(_i32)
    pk = jnp.bitwise_or(jnp.left_shift(src, 14), dst)
    pk3 = jnp.pad(pk.reshape(_NW, _EPW), ((0, 0), (0, _EPWP - _EPW)),
                  constant_values=_PAD_DST).reshape(_NW, _NCH, _C)

    deg16 = _deg_kernel()(pk3)

    h = _tc_call(_bn_body, jax.ShapeDtypeStruct((N, D), _f32))(x, gamma2,
                                                               beta2)

    isd, g1a, g1b, y0 = _tc_call(
        _prep_body,
        (jax.ShapeDtypeStruct((N, 1), _f32),
         jax.ShapeDtypeStruct((N, DH), _bf16),
         jax.ShapeDtypeStruct((N, DH), _bf16),
         jax.ShapeDtypeStruct((N, D), _f32)))(deg16, h, W[0], W[2])

    s1a, s1b = _segsum_kernel()(g1a, g1b, pk3)

    g2a, g2b, y01 = _tc_call(
        _mid_body,
        (jax.ShapeDtypeStruct((N, DH), _bf16),
         jax.ShapeDtypeStruct((N, DH), _bf16),
         jax.ShapeDtypeStruct((N, D), _f32)))(s1a, s1b, isd, y0, W[1])

    s2a, s2b = _segsum_kernel()(g2a, g2b, pk3)

    out = _tc_call(
        _fin_body, jax.ShapeDtypeStruct((N, D), _f32))(s2a, s2b, isd, y01,
                                                       W[2], b2)
    return out


# speedup vs baseline: 1.0527x; 1.0527x over previous
import functools

import jax
import jax.numpy as jnp
from jax import lax
from jax.experimental import pallas as pl
from jax.experimental.pallas import tpu as pltpu
from jax.experimental.pallas import tpu_sc as plsc

N = 10000
E = 320000
D = 128
DH = D // 2

_f32 = jnp.float32
_i32 = jnp.int32
_bf16 = jnp.bfloat16

_NC = 2
_NS = 16
_NW = _NC * _NS
_EPW = E // _NW
_C = 128
_NCH = 79
_EPWP = _NCH * _C
_NP = 10112
_RPT = _NP // _NS
_PAD_DST = _NP - 1
_ZR = 158
_QB = 6


def _bnprep_body(x_ref, deg16_ref, gamma_ref, beta_ref,
                 isd_ref, h_ref, g1a_ref, g1b_ref):
    x = x_ref[...]
    mean = jnp.mean(x, axis=0, keepdims=True)
    xc = x - mean
    var = jnp.mean(xc * xc, axis=0, keepdims=True)
    h = xc * jax.lax.rsqrt(var + 1e-5) * gamma_ref[...] + beta_ref[...]
    h_ref[...] = h
    deg = deg16_ref[0, :N, 0:1] + deg16_ref[1, :N, 0:1]
    isd = jnp.where(deg > 0, jax.lax.rsqrt(deg), 0.0)
    isd_ref[...] = isd
    g1 = (isd * h).astype(jnp.bfloat16)
    g1a_ref[...] = g1[:, :DH]
    g1b_ref[...] = g1[:, DH:]


def _y0_body(h_ref, w0_ref, w2_ref, y0_ref):
    y0_ref[...] = jnp.dot(h_ref[...], w0_ref[...] - w2_ref[...],
                          preferred_element_type=jnp.float32)


def _midlite_body(s1a_ref, s1b_ref, isd_ref, tx1_ref, g2a_ref, g2b_ref):
    isd = isd_ref[...]
    s1 = jnp.concatenate(
        [s1a_ref[0, :N].astype(jnp.float32)
         + s1a_ref[1, :N].astype(jnp.float32),
         s1b_ref[0, :N].astype(jnp.float32)
         + s1b_ref[1, :N].astype(jnp.float32)], axis=1)
    tx1 = -isd * s1
    tx1_ref[...] = tx1
    g2 = (isd * tx1).astype(jnp.bfloat16)
    g2a_ref[...] = g2[:, :DH]
    g2b_ref[...] = g2[:, DH:]


def _acc1_body(y0_ref, tx1_ref, w1_ref, y01_ref):
    y01_ref[...] = y0_ref[...] + jnp.dot(tx1_ref[...], w1_ref[...],
                                         preferred_element_type=jnp.float32)


def _fin_body(s2a_ref, s2b_ref, isd_ref, y01_ref, w2_ref, b_ref, out_ref):
    s2 = jnp.concatenate(
        [s2a_ref[0, :N].astype(jnp.float32)
         + s2a_ref[1, :N].astype(jnp.float32),
         s2b_ref[0, :N].astype(jnp.float32)
         + s2b_ref[1, :N].astype(jnp.float32)], axis=1)
    tx2s = (-2.0 * isd_ref[...]) * s2
    acc = y01_ref[...] + jnp.dot(tx2s, w2_ref[...],
                                 preferred_element_type=jnp.float32)
    out_ref[...] = jnp.maximum(acc + b_ref[...], 0.0)


def _tc_call(body, out_shapes):
    return pl.pallas_call(body, out_shape=out_shapes)


def _unpack_dst(pk):
    return jnp.bitwise_and(pk, 16383)


def _unpack_src(pk):
    return lax.shift_right_logical(pk, 14)


def _deg_body(pk_hbm, out_hbm, pkv, didx_v, ones_v, zbuf_v, acc_sh):
    cid = lax.axis_index("c")
    sid = lax.axis_index("s")
    wid = sid * _NC + cid
    r0 = sid * _RPT

    pltpu.sync_copy(pk_hbm.at[wid], pkv)

    def fill1(i, c):
        ones_v[i] = jnp.full((16,), 1.0, _f32)
        return c

    lax.fori_loop(0, _C, fill1, 0)

    def fill0(i, c):
        zbuf_v[i] = jnp.zeros((16,), _f32)
        return c

    lax.fori_loop(0, _RPT, fill0, 0)

    def unpack(i, c):
        for k in range(_C // 16):
            pk = pkv[i, pl.ds(k * 16, 16)]
            didx_v[i, pl.ds(k * 16, 16)] = _unpack_dst(pk)
        return c

    lax.fori_loop(0, _NCH, unpack, 0)

    pltpu.sync_copy(zbuf_v, acc_sh.at[pl.ds(r0, _RPT)])
    plsc.subcore_barrier()

    def chunk(j, c):
        pltpu.sync_copy(ones_v, acc_sh.at[didx_v.at[j]], add=True)
        return c

    lax.fori_loop(0, _NCH, chunk, 0)
    plsc.subcore_barrier()
    pltpu.sync_copy(acc_sh.at[pl.ds(r0, _RPT)],
                    out_hbm.at[cid, pl.ds(r0, _RPT)])


def _segsum_body(ga_hbm, gb_hbm, pk_hbm, outa_hbm, outb_hbm,
                 pkv, sidx_v, didx_v, rows_v, zbuf_v,
                 ga_sh, gb_sh, acc_sh, *sems):
    gsems = sems[:_QB]
    ssems = sems[_QB:]
    cid = lax.axis_index("c")
    sid = lax.axis_index("s")
    wid = sid * _NC + cid
    r0 = sid * _RPT

    pltpu.sync_copy(pk_hbm.at[wid], pkv)

    @pl.when(sid == 0)
    def _():
        pltpu.sync_copy(ga_hbm, ga_sh)

    @pl.when(sid == 1)
    def _():
        pltpu.sync_copy(gb_hbm, gb_sh)

    def fill0(i, c):
        for k in range(DH // 32):
            zbuf_v[i, pl.ds(k * 32, 32)] = jnp.zeros((32,), _bf16)
        return c

    lax.fori_loop(0, _ZR, fill0, 0)

    def unpack(i, c):
        for k in range(_C // 16):
            pk = pkv[i, pl.ds(k * 16, 16)]
            sidx_v[i, pl.ds(k * 16, 16)] = _unpack_src(pk)
            didx_v[i, pl.ds(k * 16, 16)] = _unpack_dst(pk)
        return c

    lax.fori_loop(0, _NCH, unpack, 0)

    for q in range(_RPT // _ZR):
        pltpu.sync_copy(zbuf_v, acc_sh.at[pl.ds(r0 + q * _ZR, _ZR)])
    plsc.subcore_barrier()

    def run_half(g_sh, out_hbm, last):
        for b in range(_QB):
            pltpu.async_copy(g_sh.at[sidx_v.at[b]], rows_v.at[b], gsems[b])

        def step(j, c):
            b = lax.rem(j, _QB)
            jj = j + _QB - 2
            for bs in range(_QB):

                @pl.when(b == bs)
                def _():
                    pltpu.make_async_copy(g_sh.at[sidx_v.at[j]],
                                          rows_v.at[bs], gsems[bs]).wait()
                    pltpu.async_copy(rows_v.at[bs], acc_sh.at[didx_v.at[j]],
                                     ssems[bs], add=True)

                @pl.when((jj < _NCH) & (lax.rem(jj, _QB) == bs) & (jj >= _QB))
                def _():
                    pltpu.make_async_copy(rows_v.at[bs],
                                          acc_sh.at[didx_v.at[0]],
                                          ssems[bs]).wait()
                    pltpu.async_copy(g_sh.at[sidx_v.at[jj]], rows_v.at[bs],
                                     gsems[bs])

            return c

        lax.fori_loop(0, _NCH, step, 0)
        for b in range(_QB):
            pltpu.make_async_copy(rows_v.at[b], acc_sh.at[didx_v.at[0]],
                                  ssems[b]).wait()

        plsc.subcore_barrier()
        pltpu.sync_copy(acc_sh.at[pl.ds(r0, _RPT)],
                        out_hbm.at[cid, pl.ds(r0, _RPT)])
        if not last:
            for q in range(_RPT // _ZR):
                pltpu.sync_copy(zbuf_v, acc_sh.at[pl.ds(r0 + q * _ZR, _ZR)])
            plsc.subcore_barrier()

    run_half(ga_sh, outa_hbm, False)
    run_half(gb_sh, outb_hbm, True)


def _sc_mesh():
    return plsc.VectorSubcoreMesh(core_axis_name="c", subcore_axis_name="s")


@functools.lru_cache(maxsize=None)
def _deg_kernel():
    return pl.kernel(
        _deg_body,
        out_type=jax.ShapeDtypeStruct((_NC, _NP, 16), _f32),
        mesh=_sc_mesh(),
        compiler_params=pltpu.CompilerParams(use_tc_tiling_on_sc=False),
        scratch_types=[
            pltpu.VMEM((_NCH, _C), _i32),
            pltpu.VMEM((_NCH, _C), _i32),
            pltpu.VMEM((_C, 16), _f32),
            pltpu.VMEM((_RPT, 16), _f32),
            pltpu.VMEM_SHARED((_NP, 16), _f32),
        ])


@functools.lru_cache(maxsize=None)
def _segsum_kernel():
    return pl.kernel(
        _segsum_body,
        out_type=(jax.ShapeDtypeStruct((_NC, _NP, DH), _bf16),
                  jax.ShapeDtypeStruct((_NC, _NP, DH), _bf16)),
        mesh=_sc_mesh(),
        compiler_params=pltpu.CompilerParams(use_tc_tiling_on_sc=False),
        scratch_types=[
            pltpu.VMEM((_NCH, _C), _i32),
            pltpu.VMEM((_NCH, _C), _i32),
            pltpu.VMEM((_NCH, _C), _i32),
            pltpu.VMEM((_QB, _C, DH), _bf16),
            pltpu.VMEM((_ZR, DH), _bf16),
            pltpu.VMEM_SHARED((N, DH), _bf16),
            pltpu.VMEM_SHARED((N, DH), _bf16),
            pltpu.VMEM_SHARED((_NP, DH), _bf16),
        ] + [pltpu.SemaphoreType.DMA] * (2 * _QB))


@jax.jit
def kernel(x, edge_index, W, b, gamma, beta):
    gamma2 = gamma.reshape(1, D)
    beta2 = beta.reshape(1, D)
    b2 = b.reshape(1, D)

    src = edge_index[0].astype(_i32)
    dst = edge_index[1].astype(_i32)
    pk = jnp.bitwise_or(jnp.left_shift(src, 14), dst)
    pk3 = jnp.pad(pk.reshape(_NW, _EPW), ((0, 0), (0, _EPWP - _EPW)),
                  constant_values=_PAD_DST).reshape(_NW, _NCH, _C)

    deg16 = _deg_kernel()(pk3)

    isd, h, g1a, g1b = _tc_call(
        _bnprep_body,
        (jax.ShapeDtypeStruct((N, 1), _f32),
         jax.ShapeDtypeStruct((N, D), _f32),
         jax.ShapeDtypeStruct((N, DH), _bf16),
         jax.ShapeDtypeStruct((N, DH), _bf16)))(x, deg16, gamma2, beta2)

    s1a, s1b = _segsum_kernel()(g1a, g1b, pk3)

    y0 = _tc_call(
        _y0_body, jax.ShapeDtypeStruct((N, D), _f32))(h, W[0], W[2])

    tx1, g2a, g2b = _tc_call(
        _midlite_body,
        (jax.ShapeDtypeStruct((N, D), _f32),
         jax.ShapeDtypeStruct((N, DH), _bf16),
         jax.ShapeDtypeStruct((N, DH), _bf16)))(s1a, s1b, isd)

    s2a, s2b = _segsum_kernel()(g2a, g2b, pk3)

    y01 = _tc_call(
        _acc1_body, jax.ShapeDtypeStruct((N, D), _f32))(y0, tx1, W[1])

    out = _tc_call(
        _fin_body, jax.ShapeDtypeStruct((N, D), _f32))(s2a, s2b, isd, y01,
                                                       W[2], b2)
    return out
